# Initial kernel scaffold; baseline (speedup 1.0000x reference)
import jax
import jax.numpy as jnp
from jax.experimental import pallas as pl


def kernel(structure, H, input_weight, W1, b1, W2, b2, W3, b3):
    raise NotImplementedError("write your pallas kernel here")


# speedup vs baseline: 10.6057x; 10.6057x over previous
import functools

import jax
import jax.numpy as jnp
from jax import lax
from jax.experimental import pallas as pl
from jax.experimental.pallas import tpu as pltpu
from jax.experimental.pallas import tpu_sc as plsc

N_NODES = 10000
N_EDGES = 320000
D_IN = 128

NC = 2
NS = 16
NW = NC * NS
CHUNK = 128
N_PAD = 10112
RPT = N_PAD // NS
EPT_CHUNKS = -(-N_EDGES // (NW * CHUNK))
E_PAD = NW * EPT_CHUNKS * CHUNK


def _sc_scatter(hw, zeros, src_r, dst_r, d):
    mesh = plsc.VectorSubcoreMesh(core_axis_name="c", subcore_axis_name="s")

    @functools.partial(
        pl.kernel,
        out_type=jax.ShapeDtypeStruct((NC, N_PAD, d), jnp.float32),
        mesh=mesh,
        scratch_types=[
            pltpu.VMEM((EPT_CHUNKS, CHUNK), jnp.int32),
            pltpu.VMEM((EPT_CHUNKS, CHUNK), jnp.int32),
            pltpu.VMEM((CHUNK, d), jnp.float32),
            pltpu.VMEM_SHARED((N_PAD, d), jnp.float32),
            pltpu.SemaphoreType.DMA,
        ],
        compiler_params=pltpu.CompilerParams(use_tc_tiling_on_sc=False),
    )
    def k(hw_hbm, z_hbm, src_hbm, dst_hbm, out_hbm,
          src_v, dst_v, rows_v, acc_sh, sem):
        c = lax.axis_index("c")
        s = lax.axis_index("s")
        wid = s * NC + c
        pltpu.sync_copy(src_hbm.at[wid], src_v)
        pltpu.sync_copy(dst_hbm.at[wid], dst_v)
        pltpu.sync_copy(z_hbm.at[pl.ds(s * RPT, RPT)],
                        acc_sh.at[pl.ds(s * RPT, RPT)])
        plsc.subcore_barrier()

        @pl.loop(0, EPT_CHUNKS)
        def _(j):
            pltpu.async_copy(hw_hbm.at[src_v.at[j]], rows_v, sem).wait()
            pltpu.sync_copy(rows_v, acc_sh.at[dst_v.at[j]], add=True)

        plsc.subcore_barrier()
        pltpu.sync_copy(acc_sh.at[pl.ds(s * RPT, RPT)],
                        out_hbm.at[c].at[pl.ds(s * RPT, RPT)])

    return k(hw, zeros, src_r, dst_r)


def _tc_scale(h, wcol):
    def body(h_ref, wc_ref, o_ref):
        o_ref[...] = wc_ref[...] * h_ref[...]

    return pl.pallas_call(
        body,
        out_shape=jax.ShapeDtypeStruct(h.shape, jnp.float32),
    )(h, wcol)


def _tc_layer(p, h, w_mat, b, wcol):
    def body(p_ref, h_ref, w_ref, b_ref, wc_ref, hn_ref, hwn_ref):
        x = p_ref[0] + p_ref[1] + h_ref[...]
        hn = jnp.maximum(
            jnp.dot(x, w_ref[...], preferred_element_type=jnp.float32)
            + b_ref[...], 0.0)
        hn_ref[...] = hn
        hwn_ref[...] = wc_ref[...] * hn

    d = w_mat.shape[1]
    return pl.pallas_call(
        body,
        out_shape=[
            jax.ShapeDtypeStruct((N_PAD, d), jnp.float32),
            jax.ShapeDtypeStruct((N_PAD, d), jnp.float32),
        ],
    )(p, h, w_mat, b, wcol)


def _tc_last(p, h, w_mat, b):
    def body(p_ref, h_ref, w_ref, b_ref, o_ref):
        x = p_ref[0] + p_ref[1] + h_ref[...]
        o_ref[...] = jax.nn.sigmoid(
            jnp.dot(x, w_ref[...], preferred_element_type=jnp.float32)
            + b_ref[...])

    d = w_mat.shape[1]
    return pl.pallas_call(
        body,
        out_shape=jax.ShapeDtypeStruct((N_PAD, d), jnp.float32),
    )(p, h, w_mat, b)


def kernel(structure, H, input_weight, W1, b1, W2, b2, W3, b3):
    src = structure[0]
    dst = structure[1]
    pad = E_PAD - N_EDGES
    fill = jnp.full((pad,), N_NODES, jnp.int32)
    src_r = jnp.concatenate([src, fill]).reshape(NW, EPT_CHUNKS, CHUNK)
    dst_r = jnp.concatenate([dst, fill]).reshape(NW, EPT_CHUNKS, CHUNK)

    h_pad = jnp.zeros((N_PAD, D_IN), jnp.float32).at[:N_NODES].set(H)
    wcol = jnp.zeros((N_PAD, 1), jnp.float32).at[:N_NODES, 0].set(input_weight)
    z = jnp.zeros((N_PAD, D_IN), jnp.float32)

    hw1 = _tc_scale(h_pad, wcol)
    p1 = _sc_scatter(hw1, z, src_r, dst_r, D_IN)
    h1, hw2 = _tc_layer(p1, h_pad, W1, b1.reshape(1, -1), wcol)
    p2 = _sc_scatter(hw2, z[:, :32], src_r, dst_r, 32)
    h2, hw3 = _tc_layer(p2, h1, W2, b2.reshape(1, -1), wcol)
    p3 = _sc_scatter(hw3, z[:, :16], src_r, dst_r, 16)
    out = _tc_last(p3, h2, W3, b3.reshape(1, -1))
    return out[:N_NODES]
